# baseline (speedup 1.0000x reference)
import functools

import jax
import jax.numpy as jnp
from jax import lax
from jax.experimental import pallas as pl
from jax.experimental.pallas import tpu as pltpu
from jax.experimental.pallas import tpu_sc as plsc

NUM_CLASS = 100000
N_CLS_CTX = 4
CTX_DIM = 512
TOK_LEN = 77
BATCH = 4096
PREFIX_LEN = 5
SUFFIX_LEN = 68

ROW = N_CLS_CTX * CTX_DIM
PRE_W = PREFIX_LEN * CTX_DIM
SUF_W = SUFFIX_LEN * CTX_DIM
OUT_W = TOK_LEN * CTX_DIM
CMB_W = SUF_W + PRE_W

NUM_CORES = 2
NUM_SUBCORES = 16
NW = NUM_CORES * NUM_SUBCORES
BPW = BATCH // NW
CH = 16
NCH = BPW // CH


def _body(lab_hbm, table_hbm, pfx_hbm, sfx_hbm, out_hbm,
          idx_v, stage_v, rows_v, sem_g, sem_m, sem_w, sem_e):
    wid = lax.axis_index("s") * NUM_CORES + lax.axis_index("c")
    base = wid * BPW
    pltpu.sync_copy(lab_hbm.at[pl.ds(base, BPW)], idx_v)
    pltpu.sync_copy(sfx_hbm, stage_v.at[pl.ds(0, SUF_W)])
    pltpu.sync_copy(pfx_hbm, stage_v.at[pl.ds(SUF_W, PRE_W)])

    first_pfx = pltpu.async_copy(
        stage_v.at[pl.ds(SUF_W, PRE_W)],
        out_hbm.at[pl.ds(base * OUT_W, PRE_W)], sem_e)

    def chunk(c, _):
        cb = base + c * CH

        @pl.when(c >= 1)
        def _():
            for _i in range(CH):
                pltpu.make_async_copy(
                    rows_v.at[0], out_hbm.at[pl.ds(0, ROW)], sem_m
                ).wait()

        pltpu.async_copy(
            table_hbm.at[idx_v.at[pl.ds(c * CH, CH)]], rows_v, sem_g,
        ).wait()

        for i in range(CH):
            b = cb + i
            pltpu.async_copy(
                rows_v.at[i], out_hbm.at[pl.ds(b * OUT_W + PRE_W, ROW)],
                sem_m)
            if i < CH - 1:
                pltpu.async_copy(
                    stage_v, out_hbm.at[pl.ds(b * OUT_W + PRE_W + ROW, CMB_W)],
                    sem_w)
            else:
                @pl.when(c < NCH - 1)
                def _():
                    pltpu.async_copy(
                        stage_v,
                        out_hbm.at[pl.ds(b * OUT_W + PRE_W + ROW, CMB_W)],
                        sem_w)

                @pl.when(c == NCH - 1)
                def _():
                    pltpu.async_copy(
                        stage_v.at[pl.ds(0, SUF_W)],
                        out_hbm.at[pl.ds(b * OUT_W + PRE_W + ROW, SUF_W)],
                        sem_e)

        @pl.when(c >= 1)
        def _():
            for _i in range(CH):
                pltpu.make_async_copy(
                    stage_v, out_hbm.at[pl.ds(0, CMB_W)], sem_w
                ).wait()

        return 0

    lax.fori_loop(0, NCH, chunk, 0)

    for _i in range(CH - 1):
        pltpu.make_async_copy(
            stage_v, out_hbm.at[pl.ds(0, CMB_W)], sem_w).wait()
    for _i in range(CH):
        pltpu.make_async_copy(
            rows_v.at[0], out_hbm.at[pl.ds(0, ROW)], sem_m).wait()
    first_pfx.wait()
    pltpu.make_async_copy(
        stage_v.at[pl.ds(0, SUF_W)], out_hbm.at[pl.ds(0, SUF_W)], sem_e
    ).wait()


def kernel(label, cls_ctx, token_prefix, token_suffix):
    table = cls_ctx.reshape(NUM_CLASS, ROW)
    pfx = token_prefix.reshape(PRE_W)
    sfx = token_suffix.reshape(SUF_W)
    lab = label.astype(jnp.int32)

    mesh = plsc.VectorSubcoreMesh(
        core_axis_name="c", subcore_axis_name="s",
        num_cores=NUM_CORES, num_subcores=NUM_SUBCORES,
    )
    run = functools.partial(
        pl.kernel,
        out_type=jax.ShapeDtypeStruct((BATCH * OUT_W,), jnp.float32),
        mesh=mesh,
        scratch_types=[
            pltpu.VMEM((BPW,), jnp.int32),
            pltpu.VMEM((CMB_W,), jnp.float32),
            pltpu.VMEM((CH, ROW), jnp.float32),
            pltpu.SemaphoreType.DMA,
            pltpu.SemaphoreType.DMA,
            pltpu.SemaphoreType.DMA,
            pltpu.SemaphoreType.DMA,
        ],
    )(_body)
    out = run(lab, table, pfx, sfx)
    return out.reshape(BATCH, TOK_LEN, CTX_DIM)


# speedup vs baseline: 6.7041x; 6.7041x over previous
import functools

import jax
import jax.numpy as jnp
from jax import lax
from jax.experimental import pallas as pl
from jax.experimental.pallas import tpu as pltpu
from jax.experimental.pallas import tpu_sc as plsc

NUM_CLASS = 100000
N_CLS_CTX = 4
CTX_DIM = 512
TOK_LEN = 77
BATCH = 4096
PREFIX_LEN = 5
SUFFIX_LEN = 68

NUM_CORES = 2
NUM_SUBCORES = 16
NW = NUM_CORES * NUM_SUBCORES
N_MID = N_CLS_CTX
N_BC_W = NW - N_MID
N_BC_PLANES = TOK_LEN - N_CLS_CTX
BC_EXTRA = N_BC_PLANES % N_BC_W

ROWS3 = BATCH * (CTX_DIM // 128)
GRP = 128
N_GRP = ROWS3 // GRP
BB_ROWS = 256
N_BC_DMA = ROWS3 // BB_ROWS


def _body(lab_hbm, table_hbm, bc_hbm, out_hbm,
          lab_v, idx_v, g0, g1, bb0, bb1,
          sem_g0, sem_g1, sem_m0, sem_m1, sem_w):
    w = lax.axis_index("s") * NUM_CORES + lax.axis_index("c")

    @pl.when(w < N_MID)
    def _mid():
        j = w
        t = PREFIX_LEN + j
        pltpu.sync_copy(lab_hbm, lab_v)

        def build(k, _):
            io = lax.iota(jnp.int32, 16)
            lv = lab_v[pl.ds(k * 16, 16)] * 16 + j
            lo = lv.at[io & 7].get(mode="promise_in_bounds")
            hi = lv.at[8 + (io & 7)].get(mode="promise_in_bounds")
            c01 = (io >> 3) * 4
            idx_v[pl.ds(k * 64, 16)] = lo + c01
            idx_v[pl.ds(k * 64 + 16, 16)] = lo + c01 + 8
            idx_v[pl.ds(k * 64 + 32, 16)] = hi + c01
            idx_v[pl.ds(k * 64 + 48, 16)] = hi + c01 + 8
            return 0

        lax.fori_loop(0, BATCH // 16, build, 0)

        def pair(k, _):
            g = 2 * k

            @pl.when(k >= 1)
            def _():
                pltpu.make_async_copy(
                    g0, out_hbm.at[0, pl.ds(0, GRP)], sem_m0).wait()
                pltpu.make_async_copy(
                    g1, out_hbm.at[0, pl.ds(0, GRP)], sem_m1).wait()

            c0 = pltpu.async_copy(
                table_hbm.at[idx_v.at[pl.ds(g * GRP, GRP)]], g0, sem_g0)
            c1 = pltpu.async_copy(
                table_hbm.at[idx_v.at[pl.ds((g + 1) * GRP, GRP)]], g1, sem_g1)
            c0.wait()
            pltpu.async_copy(g0, out_hbm.at[t, pl.ds(g * GRP, GRP)], sem_m0)
            c1.wait()
            pltpu.async_copy(
                g1, out_hbm.at[t, pl.ds((g + 1) * GRP, GRP)], sem_m1)
            return 0

        lax.fori_loop(0, N_GRP // 2, pair, 0)
        pltpu.make_async_copy(g0, out_hbm.at[0, pl.ds(0, GRP)], sem_m0).wait()
        pltpu.make_async_copy(g1, out_hbm.at[0, pl.ds(0, GRP)], sem_m1).wait()

    @pl.when(w >= N_MID)
    def _bcast():
        bw = w - N_MID
        n = jnp.where(bw < BC_EXTRA, 3, 2)
        start = bw * 2 + jnp.minimum(bw, BC_EXTRA)

        for s in range(3):
            bb = (bb0, bb1)[s % 2]

            @pl.when(s < n)
            def _(s=s, bb=bb):
                if s == 2:
                    def d(k, _):
                        pltpu.make_async_copy(
                            bb, out_hbm.at[0, pl.ds(0, BB_ROWS)], sem_w
                        ).wait()
                        return 0

                    lax.fori_loop(0, N_BC_DMA, d, 0)

                p = start + s
                t = jnp.where(p < PREFIX_LEN, p, p + N_CLS_CTX)
                pltpu.sync_copy(bc_hbm.at[p], bb)

                def fire(k, _):
                    pltpu.async_copy(
                        bb, out_hbm.at[t, pl.ds(k * BB_ROWS, BB_ROWS)], sem_w)
                    return 0

                lax.fori_loop(0, N_BC_DMA, fire, 0)

        def drain(k, _):
            pltpu.make_async_copy(
                bb0, out_hbm.at[0, pl.ds(0, BB_ROWS)], sem_w).wait()
            return 0

        lax.fori_loop(0, 2 * N_BC_DMA, drain, 0)


def kernel(label, cls_ctx, token_prefix, token_suffix):
    table = (cls_ctx.reshape(NUM_CLASS, N_CLS_CTX, 4, 128)
             .transpose(0, 2, 1, 3).reshape(NUM_CLASS * 16, 128))
    toks = jnp.concatenate([
        token_prefix.reshape(PREFIX_LEN, 4, 128),
        token_suffix.reshape(SUFFIX_LEN, 4, 128),
    ])
    bc = jnp.broadcast_to(
        toks[:, None, :, None, :],
        (N_BC_PLANES, BB_ROWS // 32, 4, 8, 128),
    ).reshape(N_BC_PLANES, BB_ROWS, 128)
    lab = label.astype(jnp.int32)

    mesh = plsc.VectorSubcoreMesh(
        core_axis_name="c", subcore_axis_name="s",
        num_cores=NUM_CORES, num_subcores=NUM_SUBCORES,
    )
    run = functools.partial(
        pl.kernel,
        out_type=jax.ShapeDtypeStruct((TOK_LEN, ROWS3, 128), jnp.float32),
        mesh=mesh,
        scratch_types=[
            pltpu.VMEM((BATCH,), jnp.int32),
            pltpu.VMEM((ROWS3,), jnp.int32),
            pltpu.VMEM((GRP, 128), jnp.float32),
            pltpu.VMEM((GRP, 128), jnp.float32),
            pltpu.VMEM((BB_ROWS, 128), jnp.float32),
            pltpu.VMEM((BB_ROWS, 128), jnp.float32),
            pltpu.SemaphoreType.DMA,
            pltpu.SemaphoreType.DMA,
            pltpu.SemaphoreType.DMA,
            pltpu.SemaphoreType.DMA,
            pltpu.SemaphoreType.DMA,
        ],
    )(_body)
    out = run(lab, table, bc)
    return (out.reshape(TOK_LEN, BATCH // 8, 4, 8, 128)
            .transpose(1, 3, 0, 2, 4).reshape(BATCH, TOK_LEN, CTX_DIM))


# speedup vs baseline: 7.0137x; 1.0462x over previous
import functools

import jax
import jax.numpy as jnp
from jax import lax
from jax.experimental import pallas as pl
from jax.experimental.pallas import tpu as pltpu
from jax.experimental.pallas import tpu_sc as plsc

NUM_CLASS = 100000
N_CLS_CTX = 4
CTX_DIM = 512
TOK_LEN = 77
BATCH = 4096
PREFIX_LEN = 5
SUFFIX_LEN = 68

NUM_CORES = 2
NUM_SUBCORES = 16
NW = NUM_CORES * NUM_SUBCORES
N_MID_W = 2 * N_CLS_CTX
N_BC_PLANES = TOK_LEN - N_CLS_CTX

ROWS3 = BATCH * (CTX_DIM // 128)
HROWS = ROWS3 // 2
GRP = 128
N_GRP_H = HROWS // GRP
BB_ROWS = 256
FIRES = ROWS3 // BB_ROWS
UNITS = N_BC_PLANES * 4
MID_U = 5
BC_U_BASE = N_MID_W * MID_U
N_BC_W = NW - N_MID_W
BC_Q = (UNITS - BC_U_BASE) // N_BC_W
BC_EXTRA = (UNITS - BC_U_BASE) - N_BC_W * BC_Q


def _bc_units(ustart, nu, n_slots, bb0, bb1, bc_hbm, out_hbm, sem_w):
    uend = ustart + nu
    p0 = ustart // 4

    def _bounds(s):
        p = p0 + s
        lo = jnp.maximum(ustart, 4 * p)
        hi = jnp.minimum(uend, 4 * p + 4)
        return p, lo, hi

    def _drain(bb, count16):
        def d(k, _):
            pltpu.make_async_copy(
                bb, out_hbm.at[0, pl.ds(0, BB_ROWS)], sem_w).wait()
            return 0

        lax.fori_loop(0, count16 * 16, d, 0)

    for s in range(n_slots):
        p, lo, hi = _bounds(s)
        bb = (bb0, bb1)[s % 2]

        if s >= 2:
            _, lo2, hi2 = _bounds(s - 2)
            _drain(bb, jnp.maximum(hi2 - lo2, 0))

        @pl.when(hi > lo)
        def _(s=s, p=p, lo=lo, hi=hi, bb=bb):
            t = jnp.where(p < PREFIX_LEN, p, p + N_CLS_CTX)
            pltpu.sync_copy(bc_hbm.at[p], bb)

            def fire(k, _):
                pltpu.async_copy(
                    bb, out_hbm.at[t, pl.ds(k * BB_ROWS, BB_ROWS)], sem_w)
                return 0

            lax.fori_loop((lo - 4 * p) * 16, (hi - 4 * p) * 16, fire, 0)

    for s in (n_slots - 2, n_slots - 1):
        if s < 0:
            continue
        _, lo, hi = _bounds(s)
        _drain((bb0, bb1)[s % 2], jnp.maximum(hi - lo, 0))


def _body(lab_hbm, table_hbm, bc_hbm, out_hbm,
          lab_v, idx_v, g0, g1, bb0, bb1,
          sem_g0, sem_g1, sem_m0, sem_m1, sem_w):
    w = lax.axis_index("s") * NUM_CORES + lax.axis_index("c")

    @pl.when(w < N_MID_W)
    def _mid():
        j = w >> 1
        h = w & 1
        t = PREFIX_LEN + j
        base3 = h * HROWS
        pltpu.sync_copy(lab_hbm.at[pl.ds(h * (BATCH // 2), BATCH // 2)], lab_v)

        def build(k, _):
            io = lax.iota(jnp.int32, 16)
            lv = lab_v[pl.ds(k * 16, 16)] * 16 + j
            lo = lv.at[io & 7].get(mode="promise_in_bounds")
            hi = lv.at[8 + (io & 7)].get(mode="promise_in_bounds")
            c01 = (io >> 3) * 4
            idx_v[pl.ds(k * 64, 16)] = lo + c01
            idx_v[pl.ds(k * 64 + 16, 16)] = lo + c01 + 8
            idx_v[pl.ds(k * 64 + 32, 16)] = hi + c01
            idx_v[pl.ds(k * 64 + 48, 16)] = hi + c01 + 8
            return 0

        lax.fori_loop(0, BATCH // 2 // 16, build, 0)

        def pair(k, _):
            g = 2 * k

            @pl.when(k >= 1)
            def _():
                pltpu.make_async_copy(
                    g0, out_hbm.at[0, pl.ds(0, GRP)], sem_m0).wait()
                pltpu.make_async_copy(
                    g1, out_hbm.at[0, pl.ds(0, GRP)], sem_m1).wait()

            c0 = pltpu.async_copy(
                table_hbm.at[idx_v.at[pl.ds(g * GRP, GRP)]], g0, sem_g0)
            c1 = pltpu.async_copy(
                table_hbm.at[idx_v.at[pl.ds((g + 1) * GRP, GRP)]], g1, sem_g1)
            c0.wait()
            pltpu.async_copy(
                g0, out_hbm.at[t, pl.ds(base3 + g * GRP, GRP)], sem_m0)
            c1.wait()
            pltpu.async_copy(
                g1, out_hbm.at[t, pl.ds(base3 + (g + 1) * GRP, GRP)], sem_m1)
            return 0

        lax.fori_loop(0, N_GRP_H // 2, pair, 0)
        pltpu.make_async_copy(g0, out_hbm.at[0, pl.ds(0, GRP)], sem_m0).wait()
        pltpu.make_async_copy(g1, out_hbm.at[0, pl.ds(0, GRP)], sem_m1).wait()
        _bc_units(w * MID_U, MID_U, 2, bb0, bb1, bc_hbm, out_hbm, sem_w)

    @pl.when(w >= N_MID_W)
    def _bcast():
        i = w - N_MID_W
        nu = BC_Q + jnp.where(i < BC_EXTRA, 1, 0)
        ustart = BC_U_BASE + i * BC_Q + jnp.minimum(i, BC_EXTRA)
        _bc_units(ustart, nu, 4, bb0, bb1, bc_hbm, out_hbm, sem_w)


def kernel(label, cls_ctx, token_prefix, token_suffix):
    table = (cls_ctx.reshape(NUM_CLASS, N_CLS_CTX, 4, 128)
             .transpose(0, 2, 1, 3).reshape(NUM_CLASS * 16, 128))
    toks = jnp.concatenate([
        token_prefix.reshape(PREFIX_LEN, 4, 128),
        token_suffix.reshape(SUFFIX_LEN, 4, 128),
    ])
    bc = jnp.broadcast_to(
        toks[:, None, :, None, :],
        (N_BC_PLANES, BB_ROWS // 32, 4, 8, 128),
    ).reshape(N_BC_PLANES, BB_ROWS, 128)
    lab = label.astype(jnp.int32)

    mesh = plsc.VectorSubcoreMesh(
        core_axis_name="c", subcore_axis_name="s",
        num_cores=NUM_CORES, num_subcores=NUM_SUBCORES,
    )
    run = functools.partial(
        pl.kernel,
        out_type=jax.ShapeDtypeStruct((TOK_LEN, ROWS3, 128), jnp.float32),
        mesh=mesh,
        scratch_types=[
            pltpu.VMEM((BATCH // 2,), jnp.int32),
            pltpu.VMEM((HROWS,), jnp.int32),
            pltpu.VMEM((GRP, 128), jnp.float32),
            pltpu.VMEM((GRP, 128), jnp.float32),
            pltpu.VMEM((BB_ROWS, 128), jnp.float32),
            pltpu.VMEM((BB_ROWS, 128), jnp.float32),
            pltpu.SemaphoreType.DMA,
            pltpu.SemaphoreType.DMA,
            pltpu.SemaphoreType.DMA,
            pltpu.SemaphoreType.DMA,
            pltpu.SemaphoreType.DMA,
        ],
    )(_body)
    out = run(lab, table, bc)
    return (out.reshape(TOK_LEN, BATCH // 8, 4, 8, 128)
            .transpose(1, 3, 0, 2, 4).reshape(BATCH, TOK_LEN, CTX_DIM))


# speedup vs baseline: 7.1796x; 1.0237x over previous
import functools

import jax
import jax.numpy as jnp
from jax import lax
from jax.experimental import pallas as pl
from jax.experimental.pallas import tpu as pltpu
from jax.experimental.pallas import tpu_sc as plsc

NUM_CLASS = 100000
N_CLS_CTX = 4
CTX_DIM = 512
TOK_LEN = 77
BATCH = 4096
PREFIX_LEN = 5
SUFFIX_LEN = 68

NUM_CORES = 2
NUM_SUBCORES = 16
NW = NUM_CORES * NUM_SUBCORES
N_MID_W = 2 * N_CLS_CTX
N_BC_PLANES = TOK_LEN - N_CLS_CTX

ROWS3 = BATCH * (CTX_DIM // 128)
HROWS = ROWS3 // 2
GRP = 128
N_GRP_H = HROWS // GRP
BB_ROWS = 256
FIRES = ROWS3 // BB_ROWS
UPP = 8
FPU = FIRES // UPP
UNITS = N_BC_PLANES * UPP
MID_U = 11
BC_U_BASE = N_MID_W * MID_U
N_BC_W = NW - N_MID_W
BC_Q = (UNITS - BC_U_BASE) // N_BC_W
BC_EXTRA = (UNITS - BC_U_BASE) - N_BC_W * BC_Q


def _bc_units(ustart, nu, n_slots, bb0, bb1, bc_hbm, out_hbm, sem_w):
    uend = ustart + nu
    p0 = ustart // UPP

    def _bounds(s):
        p = p0 + s
        lo = jnp.maximum(ustart, UPP * p)
        hi = jnp.minimum(uend, UPP * p + UPP)
        return p, lo, hi

    def _drain(bb, count16):
        def d(k, _):
            pltpu.make_async_copy(
                bb, out_hbm.at[0, pl.ds(0, BB_ROWS)], sem_w).wait()
            return 0

        lax.fori_loop(0, count16 * FPU, d, 0)

    for s in range(n_slots):
        p, lo, hi = _bounds(s)
        bb = (bb0, bb1)[s % 2]

        if s >= 2:
            _, lo2, hi2 = _bounds(s - 2)
            _drain(bb, jnp.maximum(hi2 - lo2, 0))

        @pl.when(hi > lo)
        def _(s=s, p=p, lo=lo, hi=hi, bb=bb):
            t = jnp.where(p < PREFIX_LEN, p, p + N_CLS_CTX)
            pltpu.sync_copy(bc_hbm.at[p], bb)

            def fire(k, _):
                pltpu.async_copy(
                    bb, out_hbm.at[t, pl.ds(k * BB_ROWS, BB_ROWS)], sem_w)
                return 0

            lax.fori_loop((lo - UPP * p) * FPU, (hi - UPP * p) * FPU, fire, 0)

    for s in (n_slots - 2, n_slots - 1):
        if s < 0:
            continue
        _, lo, hi = _bounds(s)
        _drain((bb0, bb1)[s % 2], jnp.maximum(hi - lo, 0))


def _body(lab_hbm, table_hbm, bc_hbm, out_hbm,
          lab_v, idx_v, g0, g1, bb0, bb1,
          sem_g0, sem_g1, sem_m0, sem_m1, sem_w):
    w = lax.axis_index("s") * NUM_CORES + lax.axis_index("c")

    @pl.when(w < N_MID_W)
    def _mid():
        j = w >> 1
        h = w & 1
        t = PREFIX_LEN + j
        base3 = h * HROWS
        pltpu.sync_copy(lab_hbm.at[pl.ds(h * (BATCH // 2), BATCH // 2)], lab_v)

        def build(k, _):
            io = lax.iota(jnp.int32, 16)
            lv = lab_v[pl.ds(k * 16, 16)] * 16 + j
            lo = lv.at[io & 7].get(mode="promise_in_bounds")
            hi = lv.at[8 + (io & 7)].get(mode="promise_in_bounds")
            c01 = (io >> 3) * 4
            idx_v[pl.ds(k * 64, 16)] = lo + c01
            idx_v[pl.ds(k * 64 + 16, 16)] = lo + c01 + 8
            idx_v[pl.ds(k * 64 + 32, 16)] = hi + c01
            idx_v[pl.ds(k * 64 + 48, 16)] = hi + c01 + 8
            return 0

        lax.fori_loop(0, BATCH // 2 // 16, build, 0)

        def pair(k, _):
            g = 2 * k

            @pl.when(k >= 1)
            def _():
                pltpu.make_async_copy(
                    g0, out_hbm.at[0, pl.ds(0, GRP)], sem_m0).wait()
                pltpu.make_async_copy(
                    g1, out_hbm.at[0, pl.ds(0, GRP)], sem_m1).wait()

            c0 = pltpu.async_copy(
                table_hbm.at[idx_v.at[pl.ds(g * GRP, GRP)]], g0, sem_g0)
            c1 = pltpu.async_copy(
                table_hbm.at[idx_v.at[pl.ds((g + 1) * GRP, GRP)]], g1, sem_g1)
            c0.wait()
            pltpu.async_copy(
                g0, out_hbm.at[t, pl.ds(base3 + g * GRP, GRP)], sem_m0)
            c1.wait()
            pltpu.async_copy(
                g1, out_hbm.at[t, pl.ds(base3 + (g + 1) * GRP, GRP)], sem_m1)
            return 0

        lax.fori_loop(0, N_GRP_H // 2, pair, 0)
        pltpu.make_async_copy(g0, out_hbm.at[0, pl.ds(0, GRP)], sem_m0).wait()
        pltpu.make_async_copy(g1, out_hbm.at[0, pl.ds(0, GRP)], sem_m1).wait()
        _bc_units(w * MID_U, MID_U, 3, bb0, bb1, bc_hbm, out_hbm, sem_w)

    @pl.when(w >= N_MID_W)
    def _bcast():
        i = w - N_MID_W
        nu = BC_Q + jnp.where(i < BC_EXTRA, 1, 0)
        ustart = BC_U_BASE + i * BC_Q + jnp.minimum(i, BC_EXTRA)
        _bc_units(ustart, nu, 4, bb0, bb1, bc_hbm, out_hbm, sem_w)


def kernel(label, cls_ctx, token_prefix, token_suffix):
    table = (cls_ctx.reshape(NUM_CLASS, N_CLS_CTX, 4, 128)
             .transpose(0, 2, 1, 3).reshape(NUM_CLASS * 16, 128))
    toks = jnp.concatenate([
        token_prefix.reshape(PREFIX_LEN, 4, 128),
        token_suffix.reshape(SUFFIX_LEN, 4, 128),
    ])
    bc = jnp.broadcast_to(
        toks[:, None, :, None, :],
        (N_BC_PLANES, BB_ROWS // 32, 4, 8, 128),
    ).reshape(N_BC_PLANES, BB_ROWS, 128)
    lab = label.astype(jnp.int32)

    mesh = plsc.VectorSubcoreMesh(
        core_axis_name="c", subcore_axis_name="s",
        num_cores=NUM_CORES, num_subcores=NUM_SUBCORES,
    )
    run = functools.partial(
        pl.kernel,
        out_type=jax.ShapeDtypeStruct((TOK_LEN, ROWS3, 128), jnp.float32),
        mesh=mesh,
        scratch_types=[
            pltpu.VMEM((BATCH // 2,), jnp.int32),
            pltpu.VMEM((HROWS,), jnp.int32),
            pltpu.VMEM((GRP, 128), jnp.float32),
            pltpu.VMEM((GRP, 128), jnp.float32),
            pltpu.VMEM((BB_ROWS, 128), jnp.float32),
            pltpu.VMEM((BB_ROWS, 128), jnp.float32),
            pltpu.SemaphoreType.DMA,
            pltpu.SemaphoreType.DMA,
            pltpu.SemaphoreType.DMA,
            pltpu.SemaphoreType.DMA,
            pltpu.SemaphoreType.DMA,
        ],
    )(_body)
    out = run(lab, table, bc)
    return (out.reshape(TOK_LEN, BATCH // 8, 4, 8, 128)
            .transpose(1, 3, 0, 2, 4).reshape(BATCH, TOK_LEN, CTX_DIM))
